# Initial kernel scaffold; baseline (speedup 1.0000x reference)
import jax
import jax.numpy as jnp
from jax.experimental import pallas as pl


def kernel(inputs, embedding):
    raise NotImplementedError("write your pallas kernel here")


# speedup vs baseline: 1.1576x; 1.1576x over previous
import functools

import jax
import jax.numpy as jnp
from jax import lax
from jax.experimental import pallas as pl
from jax.experimental.pallas import tpu as pltpu
from jax.experimental.pallas import tpu_sc as plsc

_EMB = 128
_NC = 2
_NS = 16
_NW = _NC * _NS
_CHUNK = 128


@functools.partial(jax.jit, static_argnums=(2, 3))
def _gather_rows(idx, table, n_rows, nchunk):
    rows_per_w = n_rows // _NW

    mesh = plsc.VectorSubcoreMesh(core_axis_name="c", subcore_axis_name="s")

    @functools.partial(
        pl.kernel,
        out_type=jax.ShapeDtypeStruct((n_rows, _EMB), jnp.float32),
        mesh=mesh,
        scratch_types=[
            pltpu.VMEM((nchunk, _CHUNK), jnp.int32),
            pltpu.VMEM((_CHUNK, _EMB), jnp.float32),
            pltpu.SemaphoreType.DMA,
        ],
    )
    def body(idx_hbm, table_hbm, out_hbm, idx_v, rows_v, sem):
        wid = lax.axis_index("s") * _NC + lax.axis_index("c")
        pltpu.sync_copy(idx_hbm.at[wid], idx_v)
        base = wid * rows_per_w

        def chunk(j, carry):
            pltpu.async_copy(table_hbm.at[idx_v.at[j]], rows_v, sem).wait()
            pltpu.sync_copy(rows_v, out_hbm.at[pl.ds(base + j * _CHUNK, _CHUNK)])
            return carry

        lax.fori_loop(0, nchunk, chunk, 0)

    return body(idx, table)


def kernel(inputs, embedding):
    b, f = inputs.shape
    n_rows = b * f
    rows_per_w = n_rows // _NW
    nchunk = rows_per_w // _CHUNK
    idx = inputs.reshape(_NW, nchunk, _CHUNK).astype(jnp.int32)
    out = _gather_rows(idx, embedding, n_rows, nchunk)
    return out.reshape(b, f, _EMB)


# speedup vs baseline: 1.2816x; 1.1071x over previous
import functools

import jax
import jax.numpy as jnp
from jax import lax
from jax.experimental import pallas as pl
from jax.experimental.pallas import tpu as pltpu
from jax.experimental.pallas import tpu_sc as plsc

_EMB = 128
_NC = 2
_NS = 16
_NW = _NC * _NS
_CHUNK = 128


_NBUF = 2


@functools.partial(jax.jit, static_argnums=(2, 3))
def _gather_rows(idx, table, n_rows, nchunk):
    rows_per_w = n_rows // _NW

    mesh = plsc.VectorSubcoreMesh(core_axis_name="c", subcore_axis_name="s")

    @functools.partial(
        pl.kernel,
        out_type=jax.ShapeDtypeStruct((n_rows, _EMB), jnp.float32),
        mesh=mesh,
        scratch_types=[
            pltpu.VMEM((nchunk, _CHUNK), jnp.int32),
            pltpu.VMEM((_NBUF, _CHUNK, _EMB), jnp.float32),
            [pltpu.SemaphoreType.DMA] * _NBUF,
            [pltpu.SemaphoreType.DMA] * _NBUF,
        ],
    )
    def body(idx_hbm, table_hbm, out_hbm, idx_v, rows_v, gsems, wsems):
        wid = lax.axis_index("s") * _NC + lax.axis_index("c")
        pltpu.sync_copy(idx_hbm.at[wid], idx_v)
        base = wid * rows_per_w

        def g_start(j, b):
            pltpu.async_copy(table_hbm.at[idx_v.at[j]], rows_v.at[b], gsems[b])

        def g_wait(j, b):
            pltpu.make_async_copy(
                table_hbm.at[idx_v.at[j]], rows_v.at[b], gsems[b]).wait()

        def w_start(j, b):
            pltpu.async_copy(
                rows_v.at[b], out_hbm.at[pl.ds(base + j * _CHUNK, _CHUNK)],
                wsems[b])

        def w_wait(j, b):
            pltpu.make_async_copy(
                rows_v.at[b], out_hbm.at[pl.ds(base + j * _CHUNK, _CHUNK)],
                wsems[b]).wait()

        for b in range(_NBUF):
            g_start(b, b)

        def grp(gi, carry):
            g = gi * _NBUF
            for b in range(_NBUF):
                j = g + b
                g_wait(j, b)
                w_start(j, b)
                w_wait(j, b)

                @pl.when(j + _NBUF < nchunk)
                def _():
                    g_start(j + _NBUF, b)
            return carry

        lax.fori_loop(0, nchunk // _NBUF, grp, 0)

    return body(idx, table)


def kernel(inputs, embedding):
    b, f = inputs.shape
    n_rows = b * f
    rows_per_w = n_rows // _NW
    nchunk = rows_per_w // _CHUNK
    idx = inputs.reshape(_NW, nchunk, _CHUNK).astype(jnp.int32)
    out = _gather_rows(idx, embedding, n_rows, nchunk)
    return out.reshape(b, f, _EMB)


# speedup vs baseline: 1.2993x; 1.0138x over previous
import functools

import jax
import jax.numpy as jnp
from jax import lax
from jax.experimental import pallas as pl
from jax.experimental.pallas import tpu as pltpu
from jax.experimental.pallas import tpu_sc as plsc

_EMB = 128
_NC = 2
_NS = 16
_NW = _NC * _NS
_CHUNK = 128


_NBUF = 4


@functools.partial(jax.jit, static_argnums=(2, 3))
def _gather_rows(idx, table, n_rows, nchunk):
    rows_per_w = n_rows // _NW

    mesh = plsc.VectorSubcoreMesh(core_axis_name="c", subcore_axis_name="s")

    @functools.partial(
        pl.kernel,
        out_type=jax.ShapeDtypeStruct((n_rows, _EMB), jnp.float32),
        mesh=mesh,
        scratch_types=[
            pltpu.VMEM((nchunk, _CHUNK), jnp.int32),
            pltpu.VMEM((_NBUF, _CHUNK, _EMB), jnp.float32),
            [pltpu.SemaphoreType.DMA] * _NBUF,
            [pltpu.SemaphoreType.DMA] * _NBUF,
        ],
    )
    def body(idx_hbm, table_hbm, out_hbm, idx_v, rows_v, gsems, wsems):
        wid = lax.axis_index("s") * _NC + lax.axis_index("c")
        pltpu.sync_copy(idx_hbm.at[wid], idx_v)
        base = wid * rows_per_w

        def g_start(j, b):
            pltpu.async_copy(table_hbm.at[idx_v.at[j]], rows_v.at[b], gsems[b])

        def g_wait(j, b):
            pltpu.make_async_copy(
                table_hbm.at[idx_v.at[j]], rows_v.at[b], gsems[b]).wait()

        def w_start(j, b):
            pltpu.async_copy(
                rows_v.at[b], out_hbm.at[pl.ds(base + j * _CHUNK, _CHUNK)],
                wsems[b])

        def w_wait(j, b):
            pltpu.make_async_copy(
                rows_v.at[b], out_hbm.at[pl.ds(base + j * _CHUNK, _CHUNK)],
                wsems[b]).wait()

        for b in range(_NBUF):
            g_start(b, b)

        def grp(gi, carry):
            g = gi * _NBUF
            for b in range(_NBUF):
                j = g + b

                @pl.when(j < nchunk)
                def _():
                    g_wait(j, b)
                    w_start(j, b)
                    w_wait(j, b)

                    @pl.when(j + _NBUF < nchunk)
                    def _():
                        g_start(j + _NBUF, b)
            return carry

        lax.fori_loop(0, -(-nchunk // _NBUF), grp, 0)

    return body(idx, table)


def kernel(inputs, embedding):
    b, f = inputs.shape
    n_rows = b * f
    rows_per_w = n_rows // _NW
    nchunk = rows_per_w // _CHUNK
    idx = inputs.reshape(_NW, nchunk, _CHUNK).astype(jnp.int32)
    out = _gather_rows(idx, embedding, n_rows, nchunk)
    return out.reshape(b, f, _EMB)


# speedup vs baseline: 2.0545x; 1.5812x over previous
import functools

import jax
import jax.numpy as jnp
from jax import lax
from jax.experimental import pallas as pl
from jax.experimental.pallas import tpu as pltpu
from jax.experimental.pallas import tpu_sc as plsc

_EMB = 128
_NC = 2
_NS = 16
_NW = _NC * _NS
_BPC = 4
_NBUF = 4


@functools.partial(jax.jit, static_argnums=(2, 3, 4))
def _gather_rows(idx, table, batch, fields, nchunk):
    batch_per_w = batch // _NW
    rows_per_chunk = _BPC * fields

    mesh = plsc.VectorSubcoreMesh(core_axis_name="c", subcore_axis_name="s")

    @functools.partial(
        pl.kernel,
        out_type=jax.ShapeDtypeStruct((batch, fields, _EMB), jnp.float32),
        mesh=mesh,
        compiler_params=pltpu.CompilerParams(use_tc_tiling_on_sc=True),
        scratch_types=[
            pltpu.VMEM((nchunk, 128), jnp.int32),
            pltpu.VMEM((_NBUF, rows_per_chunk, _EMB), jnp.float32),
            [pltpu.SemaphoreType.DMA] * _NBUF,
            [pltpu.SemaphoreType.DMA] * _NBUF,
        ],
    )
    def body(idx_hbm, table_hbm, out_hbm, idx_v, rows_v, gsems, wsems):
        wid = lax.axis_index("s") * _NC + lax.axis_index("c")
        pltpu.sync_copy(idx_hbm.at[wid], idx_v)
        batch_base = wid * batch_per_w

        def g_start(j, b):
            pltpu.async_copy(
                table_hbm.at[idx_v.at[j, pl.ds(0, rows_per_chunk)]],
                rows_v.at[b], gsems[b])

        def g_wait(j, b):
            pltpu.make_async_copy(
                table_hbm.at[idx_v.at[j, pl.ds(0, rows_per_chunk)]],
                rows_v.at[b], gsems[b]).wait()

        def w_start(j, b):
            bb = batch_base + j * _BPC
            for k in range(_BPC):
                pltpu.async_copy(
                    rows_v.at[b, pl.ds(k * fields, fields)],
                    out_hbm.at[bb + k], wsems[b])

        def w_wait(j, b):
            bb = batch_base + j * _BPC
            for k in range(_BPC):
                pltpu.make_async_copy(
                    rows_v.at[b, pl.ds(k * fields, fields)],
                    out_hbm.at[bb + k], wsems[b]).wait()

        for b in range(_NBUF):
            g_start(b, b)

        def grp(gi, carry):
            g = gi * _NBUF
            for b in range(_NBUF):
                j = g + b
                g_wait(j, b)
                w_start(j, b)
                w_wait(j, b)

                @pl.when(j + _NBUF < nchunk)
                def _():
                    g_start(j + _NBUF, b)
            return carry

        lax.fori_loop(0, nchunk // _NBUF, grp, 0)

    return body(idx, table)


def kernel(inputs, embedding):
    batch, fields = inputs.shape
    batch_per_w = batch // _NW
    nchunk = batch_per_w // _BPC
    rows_per_chunk = _BPC * fields
    idx = inputs.astype(jnp.int32).reshape(_NW, nchunk, rows_per_chunk)
    idx = jnp.pad(idx, ((0, 0), (0, 0), (0, 128 - rows_per_chunk)))
    return _gather_rows(idx, embedding, batch, fields, nchunk)
